# baseline (speedup 1.0000x reference)
import functools

import jax
import jax.numpy as jnp
from jax import lax
from jax.experimental import pallas as pl
from jax.experimental.pallas import tpu as pltpu
from jax.experimental.pallas import tpu_sc as plsc

N_FACTORS = 32
BATCH = 16384
NC = 2
NS = 16
L = 16
NW = NC * NS
BPW = BATCH // NW
P = 128
PG = P // L


def _body(users_hbm, items_hbm, ue_hbm, ie_hbm, ub_hbm, ib_hbm, out_hbm,
          uidx_v, iidx_v, ue_v, ie_v, ub_v, ib_v, out_v, sem):
    wid = lax.axis_index("s") * NC + lax.axis_index("c")
    base = wid * BPW

    pltpu.sync_copy(users_hbm.at[pl.ds(base, BPW)], uidx_v)
    pltpu.sync_copy(items_hbm.at[pl.ds(base, BPW)], iidx_v)

    lane = lax.iota(jnp.int32, L)

    for h in range(BPW // P):
        offs = h * P

        def issue(g, carry):
            uvec = uidx_v[pl.ds(offs + g * L, L)]
            ivec = iidx_v[pl.ds(offs + g * L, L)]
            for j in range(L):
                u = uvec[j]
                it = ivec[j]
                p = g * L + j
                pltpu.async_copy(ue_hbm.at[pl.ds(u, 1)], ue_v.at[pl.ds(p, 1)],
                                 sem)
                pltpu.async_copy(ie_hbm.at[pl.ds(it, 1)], ie_v.at[pl.ds(p, 1)],
                                 sem)
                pltpu.async_copy(ub_hbm.at[pl.ds(u, 1)], ub_v.at[pl.ds(p, 1)],
                                 sem)
                pltpu.async_copy(ib_hbm.at[pl.ds(it, 1)], ib_v.at[pl.ds(p, 1)],
                                 sem)
            return carry

        lax.fori_loop(0, PG, issue, 0)

        pltpu.make_async_copy(ue_hbm.at[pl.ds(0, P)], ue_v, sem).wait()
        pltpu.make_async_copy(ie_hbm.at[pl.ds(0, P)], ie_v, sem).wait()
        pltpu.make_async_copy(ub_hbm.at[pl.ds(0, P)], ub_v, sem).wait()
        pltpu.make_async_copy(ib_hbm.at[pl.ds(0, P)], ib_v, sem).wait()

        def group(g, carry):
            acc = jnp.zeros((L,), jnp.float32)
            for j in range(L):
                p = g * L + j
                a0 = ue_v[p, pl.ds(0, L)]
                a1 = ue_v[p, pl.ds(L, L)]
                b0 = ie_v[p, pl.ds(0, L)]
                b1 = ie_v[p, pl.ds(L, L)]
                s = lax.reduce_sum(a0 * b0 + a1 * b1, axes=(0,))
                acc = jnp.where(lane == j, s, acc)
            rows = g * L + lane
            zeros = jnp.zeros((L,), jnp.int32)
            acc = acc + plsc.load_gather(ub_v, [rows, zeros])
            acc = acc + plsc.load_gather(ib_v, [rows, zeros])
            q = offs + g * L
            out_v[q // 128, pl.ds(q % 128, L)] = acc
            return carry

        lax.fori_loop(0, PG, group, 0)

    pltpu.sync_copy(out_v, out_hbm.at[pl.ds(wid * (BPW // 128), BPW // 128), :])


@jax.jit
def kernel(users, items, user_embeddings, item_embeddings, user_biases, item_biases):
    mesh = plsc.VectorSubcoreMesh(
        core_axis_name="c", subcore_axis_name="s", num_cores=NC, num_subcores=NS
    )
    run = functools.partial(
        pl.kernel,
        out_type=jax.ShapeDtypeStruct((BATCH // 128, 128), jnp.float32),
        mesh=mesh,
        compiler_params=pltpu.CompilerParams(
            needs_layout_passes=False, use_tc_tiling_on_sc=True
        ),
        scratch_types=[
            pltpu.VMEM((BPW,), jnp.int32),
            pltpu.VMEM((BPW,), jnp.int32),
            pltpu.VMEM((P, N_FACTORS), jnp.float32),
            pltpu.VMEM((P, N_FACTORS), jnp.float32),
            pltpu.VMEM((P, 1), jnp.float32),
            pltpu.VMEM((P, 1), jnp.float32),
            pltpu.VMEM((BPW // 128, 128), jnp.float32),
            pltpu.SemaphoreType.DMA,
        ],
    )(_body)
    out = run(users.astype(jnp.int32), items.astype(jnp.int32),
              user_embeddings, item_embeddings, user_biases, item_biases)
    return out.reshape(BATCH)


# speedup vs baseline: 1.1398x; 1.1398x over previous
import functools

import jax
import jax.numpy as jnp
from jax import lax
from jax.experimental import pallas as pl
from jax.experimental.pallas import tpu as pltpu
from jax.experimental.pallas import tpu_sc as plsc

N_FACTORS = 32
BATCH = 16384
NC = 2
NS = 16
L = 16
NW = NC * NS
BPW = BATCH // NW
CH = 128
NCH = BPW // CH


def _body(users_hbm, items_hbm, ue_hbm, ie_hbm, ub_hbm, ib_hbm, out_hbm,
          uidx_v, iidx_v, ue_v, ie_v, ub_v, ib_v, out_v, sem):
    wid = lax.axis_index("s") * NC + lax.axis_index("c")
    base = wid * BPW

    for i in range(NCH):
        pltpu.sync_copy(users_hbm.at[pl.ds(base + i * CH, CH)], uidx_v.at[i])
        pltpu.sync_copy(items_hbm.at[pl.ds(base + i * CH, CH)], iidx_v.at[i])

    copies = []
    for i in range(NCH):
        sl = pl.ds(i * CH, CH)
        copies.append(pltpu.async_copy(ue_hbm.at[uidx_v.at[i]], ue_v.at[sl], sem))
        copies.append(pltpu.async_copy(ie_hbm.at[iidx_v.at[i]], ie_v.at[sl], sem))
        copies.append(pltpu.async_copy(ub_hbm.at[uidx_v.at[i]], ub_v.at[sl], sem))
        copies.append(pltpu.async_copy(ib_hbm.at[iidx_v.at[i]], ib_v.at[sl], sem))
    for c in copies:
        c.wait()

    lane = lax.iota(jnp.int32, L)

    def group(g, carry):
        rows = g * L + lane
        accs = [None] * 4
        for j in range(N_FACTORS):
            col = jnp.full((L,), j, jnp.int32)
            a = plsc.load_gather(ue_v, [rows, col])
            b = plsc.load_gather(ie_v, [rows, col])
            p = a * b
            k = j % 4
            accs[k] = p if accs[k] is None else accs[k] + p
        bias = ub_v[pl.ds(g * L, L)] + ib_v[pl.ds(g * L, L)]
        out_v[pl.ds(g * L, L)] = (accs[0] + accs[1]) + (accs[2] + accs[3]) + bias
        return carry

    lax.fori_loop(0, BPW // L, group, 0, unroll=2)

    pltpu.sync_copy(out_v, out_hbm.at[pl.ds(base, BPW)])


@jax.jit
def kernel(users, items, user_embeddings, item_embeddings, user_biases, item_biases):
    mesh = plsc.VectorSubcoreMesh(
        core_axis_name="c", subcore_axis_name="s", num_cores=NC, num_subcores=NS
    )
    run = functools.partial(
        pl.kernel,
        out_type=jax.ShapeDtypeStruct((BATCH,), jnp.float32),
        mesh=mesh,
        compiler_params=pltpu.CompilerParams(
            needs_layout_passes=False, use_tc_tiling_on_sc=False
        ),
        scratch_types=[
            pltpu.VMEM((NCH, CH), jnp.int32),
            pltpu.VMEM((NCH, CH), jnp.int32),
            pltpu.VMEM((BPW, N_FACTORS), jnp.float32),
            pltpu.VMEM((BPW, N_FACTORS), jnp.float32),
            pltpu.VMEM((BPW,), jnp.float32),
            pltpu.VMEM((BPW,), jnp.float32),
            pltpu.VMEM((BPW,), jnp.float32),
            pltpu.SemaphoreType.DMA,
        ],
    )(_body)
    return run(users.astype(jnp.int32), items.astype(jnp.int32),
               user_embeddings, item_embeddings,
               user_biases.reshape(-1), item_biases.reshape(-1))
